# baseline (speedup 1.0000x reference)
import functools

import jax
import jax.numpy as jnp
from jax import lax
from jax.experimental import pallas as pl
from jax.experimental.pallas import tpu as pltpu
from jax.experimental.pallas import tpu_sc as plsc

NPTS = 512
L = 16


def _fps_body(nb, pos_hbm, out_hbm, xv, yv, zv, dist, outv, stage, pbuf, shared):
    n = xv.shape[0]
    half = dist.shape[0]
    s = lax.axis_index("s")
    c = lax.axis_index("c")
    b = c * (nb // 2) + lax.rem(s, 8)
    h = s // 8
    partner = lax.rem(s + 8, 16)
    base = h * half

    pltpu.sync_copy(pos_hbm.at[pl.ds(pl.multiple_of((0 * nb + b) * n, n), n)], xv)
    pltpu.sync_copy(pos_hbm.at[pl.ds(pl.multiple_of((1 * nb + b) * n, n), n)], yv)
    pltpu.sync_copy(pos_hbm.at[pl.ds(pl.multiple_of((2 * nb + b) * n, n), n)], zv)

    lanes = lax.iota(jnp.int32, L)

    @plsc.parallel_loop(0, half, L, unroll=8)
    def init_j(j):
        dist[pl.ds(pl.multiple_of(j, L), L)] = jnp.full((L,), 1e10, jnp.float32)

    def iter_i(t, carry):
        f, picks = carry
        fv = jnp.full((L,), f, jnp.int32)
        picks = jnp.where(lanes == t, fv, picks)
        cx = plsc.load_gather(xv, [fv])
        cy = plsc.load_gather(yv, [fv])
        cz = plsc.load_gather(zv, [fv])

        carry0 = (
            jnp.full((L,), -1.0, jnp.float32),
            jnp.zeros((L,), jnp.int32),
        )

        @plsc.parallel_loop(0, half, L, unroll=16, carry=carry0)
        def chunk(j, mc):
            mx, mi = mc
            ds_l = pl.ds(pl.multiple_of(j, L), L)
            ds_g = pl.ds(pl.multiple_of(base + j, L), L)
            dx = xv[ds_g] - cx
            dy = yv[ds_g] - cy
            dz = zv[ds_g] - cz
            d2 = dx * dx + (dy * dy + dz * dz)
            dv = jnp.minimum(dist[ds_l], d2)
            dist[ds_l] = dv
            idxs = jnp.full((L,), base + j, jnp.int32) + lanes
            gt = dv > mx
            return jnp.where(gt, dv, mx), jnp.where(gt, idxs, mi)

        mx, mi = chunk
        m = jnp.max(mx)
        cand = jnp.where(mx == m, mi, jnp.int32(n))
        li = jnp.min(cand)

        mvec = jnp.full((L,), m, jnp.float32)
        stage[...] = jnp.where(
            lanes == 0,
            plsc.bitcast(mvec, jnp.int32),
            jnp.full((L,), li, jnp.int32),
        )
        pltpu.sync_copy(stage, shared.at[pl.ds(pl.multiple_of(s * L, L), L)])
        plsc.subcore_barrier()
        pltpu.sync_copy(shared.at[pl.ds(pl.multiple_of(partner * L, L), L)], pbuf)
        pv = pbuf[...]
        pm = plsc.bitcast(pv, jnp.float32)[0]
        pi = pv[1]
        f_new = jnp.where(
            pm > m, pi, jnp.where(pm == m, jnp.minimum(li, pi), li)
        )
        return f_new, picks

    def outer_o(o, f):
        f, picks = lax.fori_loop(0, L, iter_i, (f, jnp.zeros((L,), jnp.int32)))
        outv[pl.ds(pl.multiple_of(o * L, L), L)] = picks
        return f

    lax.fori_loop(0, NPTS // L, outer_o, jnp.int32(0))

    @pl.when(h == 0)
    def _():
        pltpu.sync_copy(outv, out_hbm.at[pl.ds(pl.multiple_of(b * NPTS, NPTS), NPTS)])


def kernel(pos):
    B, N, C = pos.shape
    pos_flat = jnp.transpose(pos, (2, 0, 1)).reshape(3 * B * N)
    mesh = plsc.VectorSubcoreMesh(core_axis_name="c", subcore_axis_name="s")
    fps = pl.kernel(
        functools.partial(_fps_body, B),
        mesh=mesh,
        compiler_params=pltpu.CompilerParams(needs_layout_passes=False),
        out_type=jax.ShapeDtypeStruct((B * NPTS,), jnp.int32),
        scratch_types=[
            pltpu.VMEM((N,), jnp.float32),
            pltpu.VMEM((N,), jnp.float32),
            pltpu.VMEM((N,), jnp.float32),
            pltpu.VMEM((N // 2,), jnp.float32),
            pltpu.VMEM((NPTS,), jnp.int32),
            pltpu.VMEM((L,), jnp.int32),
            pltpu.VMEM((L,), jnp.int32),
            pltpu.VMEM_SHARED((16 * L,), jnp.int32),
        ],
    )
    return fps(pos_flat).reshape(B, NPTS)


# speedup vs baseline: 1.1661x; 1.1661x over previous
import functools

import jax
import jax.numpy as jnp
from jax import lax
from jax.experimental import pallas as pl
from jax.experimental.pallas import tpu as pltpu
from jax.experimental.pallas import tpu_sc as plsc

NPTS = 512
L = 16


def _fps_body(nb, pos_hbm, out_hbm, xv, yv, zv, dist, outv, stage, pbuf, shared):
    n = xv.shape[0]
    half = dist.shape[0]
    s = lax.axis_index("s")
    c = lax.axis_index("c")
    b = c * (nb // 2) + lax.rem(s, 8)
    h = s // 8
    partner = lax.rem(s + 8, 16)
    base = h * half

    pltpu.sync_copy(pos_hbm.at[pl.ds(pl.multiple_of((0 * nb + b) * n, n), n)], xv)
    pltpu.sync_copy(pos_hbm.at[pl.ds(pl.multiple_of((1 * nb + b) * n, n), n)], yv)
    pltpu.sync_copy(pos_hbm.at[pl.ds(pl.multiple_of((2 * nb + b) * n, n), n)], zv)

    lanes = lax.iota(jnp.int32, L)

    @plsc.parallel_loop(0, half, L, unroll=8)
    def init_j(j):
        dist[pl.ds(pl.multiple_of(j, L), L)] = jnp.full((L,), 1e10, jnp.float32)

    def iter_i(t, carry):
        f, picks = carry
        fv = jnp.full((L,), f, jnp.int32)
        picks = jnp.where(lanes == t, fv, picks)
        cx = plsc.load_gather(xv, [fv])
        cy = plsc.load_gather(yv, [fv])
        cz = plsc.load_gather(zv, [fv])

        carry0 = tuple(
            (jnp.full((L,), -1.0, jnp.float32), jnp.zeros((L,), jnp.int32))
            for _ in range(8)
        )

        @plsc.parallel_loop(0, half, 8 * L, unroll=1, carry=carry0)
        def chunk(j, acc):
            new = []
            for k in range(8):
                mx, mi = acc[k]
                off = j + k * L
                ds_l = pl.ds(pl.multiple_of(off, L), L)
                ds_g = pl.ds(pl.multiple_of(base + off, L), L)
                dx = xv[ds_g] - cx
                dy = yv[ds_g] - cy
                dz = zv[ds_g] - cz
                d2 = dx * dx + (dy * dy + dz * dz)
                dv = jnp.minimum(dist[ds_l], d2)
                dist[ds_l] = dv
                gt = dv > mx
                new.append(
                    (
                        jnp.where(gt, dv, mx),
                        jnp.where(gt, jnp.full((L,), off, jnp.int32), mi),
                    )
                )
            return tuple(new)

        def mrg(a, bb):
            va, ia = a
            vb, ib = bb
            take_b = (vb > va) | ((vb == va) & (ib < ia))
            return jnp.where(take_b, vb, va), jnp.where(take_b, ib, ia)

        pairs = list(chunk)
        while len(pairs) > 1:
            pairs = [
                mrg(pairs[i], pairs[i + 1]) for i in range(0, len(pairs), 2)
            ]
        mx, mib = pairs[0]
        mi = jnp.full((L,), base, jnp.int32) + mib + lanes
        m = jnp.max(mx)
        cand = jnp.where(mx == m, mi, jnp.int32(n))
        li = jnp.min(cand)

        mvec = jnp.full((L,), m, jnp.float32)
        stage[...] = jnp.where(
            lanes == 0,
            plsc.bitcast(mvec, jnp.int32),
            jnp.full((L,), li, jnp.int32),
        )
        pltpu.sync_copy(stage, shared.at[pl.ds(pl.multiple_of(s * L, L), L)])
        plsc.subcore_barrier()
        pltpu.sync_copy(shared.at[pl.ds(pl.multiple_of(partner * L, L), L)], pbuf)
        pv = pbuf[...]
        pm = plsc.bitcast(pv, jnp.float32)[0]
        pi = pv[1]
        f_new = jnp.where(
            pm > m, pi, jnp.where(pm == m, jnp.minimum(li, pi), li)
        )
        return f_new, picks

    def outer_o(o, f):
        f, picks = lax.fori_loop(0, L, iter_i, (f, jnp.zeros((L,), jnp.int32)))
        outv[pl.ds(pl.multiple_of(o * L, L), L)] = picks
        return f

    lax.fori_loop(0, NPTS // L, outer_o, jnp.int32(0))

    @pl.when(h == 0)
    def _():
        pltpu.sync_copy(outv, out_hbm.at[pl.ds(pl.multiple_of(b * NPTS, NPTS), NPTS)])


def kernel(pos):
    B, N, C = pos.shape
    pos_flat = jnp.transpose(pos, (2, 0, 1)).reshape(3 * B * N)
    mesh = plsc.VectorSubcoreMesh(core_axis_name="c", subcore_axis_name="s")
    fps = pl.kernel(
        functools.partial(_fps_body, B),
        mesh=mesh,
        compiler_params=pltpu.CompilerParams(needs_layout_passes=False),
        out_type=jax.ShapeDtypeStruct((B * NPTS,), jnp.int32),
        scratch_types=[
            pltpu.VMEM((N,), jnp.float32),
            pltpu.VMEM((N,), jnp.float32),
            pltpu.VMEM((N,), jnp.float32),
            pltpu.VMEM((N // 2,), jnp.float32),
            pltpu.VMEM((NPTS,), jnp.int32),
            pltpu.VMEM((L,), jnp.int32),
            pltpu.VMEM((L,), jnp.int32),
            pltpu.VMEM_SHARED((16 * L,), jnp.int32),
        ],
    )
    return fps(pos_flat).reshape(B, NPTS)
